# speedup vs baseline: 6.1711x; 6.1711x over previous
import functools

import jax
import jax.numpy as jnp
from jax.experimental import pallas as pl
from jax.experimental.pallas import tpu as pltpu

NUM_SEGMENTS = 512
BLOCK_ROWS = 2000


def _pool_body(idx_ref, x_ref, out_ref, acc_ref, cnt_ref):
    i = pl.program_id(0)
    nblk = pl.num_programs(0)

    @pl.when(i == 0)
    def _():
        acc_ref[...] = jnp.zeros_like(acc_ref)
        cnt_ref[...] = jnp.zeros_like(cnt_ref)

    x = x_ref[...]
    xm = jnp.sum(x, axis=1)
    idx = idx_ref[0, 0, :]
    seg = jax.lax.broadcasted_iota(jnp.int32, (NUM_SEGMENTS, idx.shape[0]), 0)
    onehot = (seg == idx[None, :]).astype(jnp.float32)
    acc_ref[...] += jax.lax.dot(
        onehot, xm, preferred_element_type=jnp.float32
    )
    cnt_ref[...] += jnp.sum(onehot, axis=1, keepdims=True)

    @pl.when(i == nblk - 1)
    def _():
        cnt = jnp.maximum(cnt_ref[...], 1.0)
        out_ref[...] = acc_ref[...] / (4.0 * cnt)


def kernel(node_distributions, batch_idx):
    n = node_distributions.shape[0]
    nblk = n // BLOCK_ROWS
    idx3 = batch_idx.astype(jnp.int32).reshape(nblk, 1, BLOCK_ROWS)
    out = pl.pallas_call(
        _pool_body,
        grid=(nblk,),
        in_specs=[
            pl.BlockSpec((1, 1, BLOCK_ROWS), lambda i: (i, 0, 0)),
            pl.BlockSpec((BLOCK_ROWS, 4, 128), lambda i: (i, 0, 0)),
        ],
        out_specs=pl.BlockSpec((NUM_SEGMENTS, 128), lambda i: (0, 0)),
        out_shape=jax.ShapeDtypeStruct((NUM_SEGMENTS, 128), jnp.float32),
        scratch_shapes=[
            pltpu.VMEM((NUM_SEGMENTS, 128), jnp.float32),
            pltpu.VMEM((NUM_SEGMENTS, 1), jnp.float32),
        ],
    )(idx3, node_distributions)
    return out


# speedup vs baseline: 11.6421x; 1.8865x over previous
import jax
import jax.numpy as jnp
from jax.experimental import pallas as pl
from jax.experimental.pallas import tpu as pltpu

NUM_SEGMENTS = 512
BLOCK_ROWS = 4000
WINDOW = 64


def _accum(idx4, x2, acc_ref, cnt_ref, base, width):
    seg = base + jax.lax.broadcasted_iota(jnp.int32, (width, idx4.shape[0]), 0)
    cmp = seg == idx4[None, :]
    oh = cmp.astype(jnp.bfloat16)
    m = jax.lax.dot(oh, x2, preferred_element_type=jnp.float32)
    acc_ref[pl.ds(base, width), :] += m
    cnt_ref[pl.ds(base, width), :] += jnp.sum(
        cmp.astype(jnp.float32), axis=1, keepdims=True
    )


def _pool_body(idx_ref, x_ref, out_ref, acc_ref, cnt_ref):
    i = pl.program_id(0)
    nblk = pl.num_programs(0)

    @pl.when(i == 0)
    def _():
        acc_ref[...] = jnp.zeros_like(acc_ref)
        cnt_ref[...] = jnp.zeros_like(cnt_ref)

    x2 = x_ref[...].astype(jnp.bfloat16)
    idx4 = idx_ref[0, 0, :]
    lo = jnp.min(idx4)
    hi = jnp.max(idx4)
    base = jnp.minimum((lo // 8) * 8, NUM_SEGMENTS - WINDOW)
    base = pl.multiple_of(base, 8)
    narrow = (hi - base) < WINDOW

    @pl.when(narrow)
    def _():
        _accum(idx4, x2, acc_ref, cnt_ref, base, WINDOW)

    @pl.when(jnp.logical_not(narrow))
    def _():
        _accum(idx4, x2, acc_ref, cnt_ref, 0, NUM_SEGMENTS)

    @pl.when(i == nblk - 1)
    def _():
        out_ref[...] = acc_ref[...] / jnp.maximum(cnt_ref[...], 4.0)


def kernel(node_distributions, batch_idx):
    n = node_distributions.shape[0]
    nblk = n // BLOCK_ROWS
    x4 = node_distributions.reshape(n * 4, 128)
    idx4 = jnp.repeat(batch_idx.astype(jnp.int32), 4).reshape(
        nblk, 1, BLOCK_ROWS * 4
    )
    out = pl.pallas_call(
        _pool_body,
        grid=(nblk,),
        in_specs=[
            pl.BlockSpec((1, 1, BLOCK_ROWS * 4), lambda i: (i, 0, 0)),
            pl.BlockSpec((BLOCK_ROWS * 4, 128), lambda i: (i, 0)),
        ],
        out_specs=pl.BlockSpec((NUM_SEGMENTS, 128), lambda i: (0, 0)),
        out_shape=jax.ShapeDtypeStruct((NUM_SEGMENTS, 128), jnp.float32),
        scratch_shapes=[
            pltpu.VMEM((NUM_SEGMENTS, 128), jnp.float32),
            pltpu.VMEM((NUM_SEGMENTS, 1), jnp.float32),
        ],
    )(idx4, x4)
    return out
